# Initial kernel scaffold; baseline (speedup 1.0000x reference)
import jax
import jax.numpy as jnp
from jax.experimental import pallas as pl


def kernel(focus, context, neg_indices, i_embedding, o_embedding):
    raise NotImplementedError("write your pallas kernel here")


# speedup vs baseline: 4.0411x; 4.0411x over previous
import functools

import jax
import jax.numpy as jnp
from jax import lax
from jax.experimental import pallas as pl
from jax.experimental.pallas import tpu as pltpu
from jax.experimental.pallas import tpu_sc as plsc

B = 16384
K = 20
D = 64
NC = 2
NS = 16
NW = NC * NS
PER_W = B // NW
C = 32
CHUNKS = PER_W // C
NGATHERS = C * K // 128


def _sc_dots(focus, context, negs_flat, i_emb, o_emb):
  mesh = plsc.VectorSubcoreMesh(
      core_axis_name="c", subcore_axis_name="s", num_cores=NC,
      num_subcores=NS)

  @functools.partial(
      pl.kernel,
      out_type=jax.ShapeDtypeStruct((B, K + 1), jnp.float32),
      mesh=mesh,
      compiler_params=pltpu.CompilerParams(
          needs_layout_passes=False, use_tc_tiling_on_sc=False),
      scratch_types=[
          pltpu.VMEM((PER_W,), jnp.int32),
          pltpu.VMEM((PER_W,), jnp.int32),
          pltpu.VMEM((PER_W * K,), jnp.int32),
          pltpu.VMEM((2, C, D), jnp.float32),
          pltpu.VMEM((2, C, D), jnp.float32),
          pltpu.VMEM((2, C * K, D), jnp.float32),
          pltpu.VMEM((2, C, K + 1), jnp.float32),
          pltpu.SemaphoreType.DMA,
          pltpu.SemaphoreType.DMA,
          pltpu.SemaphoreType.DMA,
          pltpu.SemaphoreType.DMA,
      ],
  )
  def k(focus_hbm, ctx_hbm, negs_hbm, iemb_hbm, oemb_hbm, out_hbm,
        idx_f, idx_c, idx_n, f_rows, c_rows, n_rows, out_buf,
        sem0, sem1, osem0, osem1):
    wid = lax.axis_index("s") * NC + lax.axis_index("c")
    base = wid * PER_W
    lanes = jnp.arange(16, dtype=jnp.int32)
    sems = (sem0, sem1)
    osems = (osem0, osem1)

    pltpu.sync_copy(focus_hbm.at[pl.ds(base, PER_W)], idx_f)
    pltpu.sync_copy(ctx_hbm.at[pl.ds(base, PER_W)], idx_c)
    pltpu.sync_copy(negs_hbm.at[pl.ds(base * K, PER_W * K)], idx_n)

    def fire(c_i, s):
      cps = [
          pltpu.async_copy(iemb_hbm.at[idx_f.at[pl.ds(c_i * C, C)]],
                           f_rows.at[s], sems[s]),
          pltpu.async_copy(oemb_hbm.at[idx_c.at[pl.ds(c_i * C, C)]],
                           c_rows.at[s], sems[s]),
      ]
      for j in range(NGATHERS):
        cps.append(pltpu.async_copy(
            oemb_hbm.at[idx_n.at[pl.ds(c_i * C * K + j * 128, 128)]],
            n_rows.at[s].at[pl.ds(j * 128, 128), :], sems[s]))
      return cps

    def compute(c_i, s):
      fr, cr, nr, ob = f_rows.at[s], c_rows.at[s], n_rows.at[s], out_buf.at[s]
      for g in range(C // 16):
        item = g * 16 + lanes
        nrow = item * K
        zero = jnp.zeros((16,), jnp.float32)

        def d_body(d, accs):
          dv = jnp.full((16,), d, jnp.int32)
          f = plsc.load_gather(fr, [item, dv])
          cvec = plsc.load_gather(cr, [item, dv])
          out = [accs[0] + f * cvec]
          fneg = -f
          for kk in range(K):
            nv = plsc.load_gather(nr, [nrow + kk, dv])
            out.append(accs[kk + 1] + fneg * nv)
          return tuple(out)

        accs = lax.fori_loop(0, D, d_body, (zero,) * (K + 1))
        for sl in range(K + 1):
          plsc.store_scatter(ob, [item, jnp.full((16,), sl, jnp.int32)],
                             accs[sl])

    out_cps = [None, None]
    cps = fire(0, 0)
    for c_i in range(CHUNKS):
      s = c_i % 2
      if c_i + 1 < CHUNKS:
        nxt = fire(c_i + 1, 1 - s)
      for cp in cps:
        cp.wait()
      if out_cps[s] is not None:
        out_cps[s].wait()
      compute(c_i, s)
      out_cps[s] = pltpu.async_copy(
          out_buf.at[s], out_hbm.at[pl.ds(base + c_i * C, C), :], osems[s])
      if c_i + 1 < CHUNKS:
        cps = nxt
    for cp in out_cps:
      cp.wait()

  return k(focus, context, negs_flat, i_emb, o_emb)


def _loss_body(x_ref, o_ref):
  x = x_ref[...]
  ls = jnp.minimum(x, 0.0) - jnp.log(1.0 + jnp.exp(-jnp.abs(x)))
  o_ref[0, 0] = -jnp.sum(ls)


def kernel(focus, context, neg_indices, i_embedding, o_embedding):
  negs_flat = neg_indices.reshape(B * K)
  dots = _sc_dots(focus, context, negs_flat, i_embedding, o_embedding)
  dots2d = dots.reshape(B * (K + 1) // 128, 128)
  loss = pl.pallas_call(
      _loss_body,
      out_shape=jax.ShapeDtypeStruct((1, 1), jnp.float32),
      in_specs=[pl.BlockSpec(memory_space=pltpu.VMEM)],
      out_specs=pl.BlockSpec(memory_space=pltpu.SMEM),
  )(dots2d)
  return loss[0, 0]


# speedup vs baseline: 5.3209x; 1.3167x over previous
import functools

import jax
import jax.numpy as jnp
from jax import lax
from jax.experimental import pallas as pl
from jax.experimental.pallas import tpu as pltpu
from jax.experimental.pallas import tpu_sc as plsc

B = 16384
K = 20
D = 64
NC = 2
NS = 16
NW = NC * NS
PER_W = B // NW
C = 32
CHUNKS = PER_W // C
NGATHERS = C * K // 128


def _sc_dots(focus, context, negs_flat, i_emb, o_emb):
  mesh = plsc.VectorSubcoreMesh(
      core_axis_name="c", subcore_axis_name="s", num_cores=NC,
      num_subcores=NS)

  @functools.partial(
      pl.kernel,
      out_type=jax.ShapeDtypeStruct((B, K + 1), jnp.float32),
      mesh=mesh,
      compiler_params=pltpu.CompilerParams(
          needs_layout_passes=False, use_tc_tiling_on_sc=False),
      scratch_types=[
          pltpu.VMEM((PER_W,), jnp.int32),
          pltpu.VMEM((PER_W,), jnp.int32),
          pltpu.VMEM((PER_W * K,), jnp.int32),
          pltpu.VMEM((2, C, D), jnp.float32),
          pltpu.VMEM((2, C, D), jnp.float32),
          pltpu.VMEM((2, C * K, D), jnp.float32),
          pltpu.VMEM((2, C, K + 1), jnp.float32),
          pltpu.SemaphoreType.DMA,
          pltpu.SemaphoreType.DMA,
          pltpu.SemaphoreType.DMA,
          pltpu.SemaphoreType.DMA,
      ],
  )
  def k(focus_hbm, ctx_hbm, negs_hbm, iemb_hbm, oemb_hbm, out_hbm,
        idx_f, idx_c, idx_n, f_rows, c_rows, n_rows, out_buf,
        sem0, sem1, osem0, osem1):
    wid = lax.axis_index("s") * NC + lax.axis_index("c")
    base = wid * PER_W
    lanes = jnp.arange(16, dtype=jnp.int32)
    sems = (sem0, sem1)
    osems = (osem0, osem1)

    pltpu.sync_copy(focus_hbm.at[pl.ds(base, PER_W)], idx_f)
    pltpu.sync_copy(ctx_hbm.at[pl.ds(base, PER_W)], idx_c)
    pltpu.sync_copy(negs_hbm.at[pl.ds(base * K, PER_W * K)], idx_n)

    def fire(c_i, s):
      cps = [
          pltpu.async_copy(iemb_hbm.at[idx_f.at[pl.ds(c_i * C, C)]],
                           f_rows.at[s], sems[s]),
          pltpu.async_copy(oemb_hbm.at[idx_c.at[pl.ds(c_i * C, C)]],
                           c_rows.at[s], sems[s]),
      ]
      for j in range(NGATHERS):
        cps.append(pltpu.async_copy(
            oemb_hbm.at[idx_n.at[pl.ds(c_i * C * K + j * 128, 128)]],
            n_rows.at[s].at[pl.ds(j * 128, 128), :], sems[s]))
      return cps

    def compute(c_i, s):
      fr, cr, nr, ob = f_rows.at[s], c_rows.at[s], n_rows.at[s], out_buf.at[s]
      for g in range(C // 16):
        item = g * 16 + lanes
        nrow = item * K
        zero = jnp.zeros((16,), jnp.float32)

        def d_body(d, accs):
          dv = (jnp.full((16,), d, jnp.int32) + lanes) & 63
          f = plsc.load_gather(fr, [item, dv])
          cvec = plsc.load_gather(cr, [item, dv])
          out = [accs[0] + f * cvec]
          fneg = -f
          for kk in range(K):
            nv = plsc.load_gather(nr, [nrow + kk, dv])
            out.append(accs[kk + 1] + fneg * nv)
          return tuple(out)

        accs = lax.fori_loop(0, D, d_body, (zero,) * (K + 1))
        for sl in range(K + 1):
          plsc.store_scatter(ob, [item, jnp.full((16,), sl, jnp.int32)],
                             accs[sl])

    out_cps = [None, None]
    cps = fire(0, 0)
    for c_i in range(CHUNKS):
      s = c_i % 2
      if c_i + 1 < CHUNKS:
        nxt = fire(c_i + 1, 1 - s)
      for cp in cps:
        cp.wait()
      if out_cps[s] is not None:
        out_cps[s].wait()
      compute(c_i, s)
      out_cps[s] = pltpu.async_copy(
          out_buf.at[s], out_hbm.at[pl.ds(base + c_i * C, C), :], osems[s])
      if c_i + 1 < CHUNKS:
        cps = nxt
    for cp in out_cps:
      cp.wait()

  return k(focus, context, negs_flat, i_emb, o_emb)


def _loss_body(x_ref, o_ref):
  x = x_ref[...]
  ls = jnp.minimum(x, 0.0) - jnp.log(1.0 + jnp.exp(-jnp.abs(x)))
  o_ref[0, 0] = -jnp.sum(ls)


def kernel(focus, context, neg_indices, i_embedding, o_embedding):
  negs_flat = neg_indices.reshape(B * K)
  dots = _sc_dots(focus, context, negs_flat, i_embedding, o_embedding)
  dots2d = dots.reshape(B * (K + 1) // 128, 128)
  loss = pl.pallas_call(
      _loss_body,
      out_shape=jax.ShapeDtypeStruct((1, 1), jnp.float32),
      in_specs=[pl.BlockSpec(memory_space=pltpu.VMEM)],
      out_specs=pl.BlockSpec(memory_space=pltpu.SMEM),
  )(dots2d)
  return loss[0, 0]
